# baseline (speedup 1.0000x reference)
import functools

import jax
import jax.numpy as jnp
from jax import lax
from jax.experimental import pallas as pl
from jax.experimental.pallas import tpu as pltpu
from jax.experimental.pallas import tpu_sc as plsc

N_ROWS = 100000
N_DIM = 64
BATCH = 4096
NC = 2
NS = 16
LANES = 16
B_W = BATCH // NS
NBLK = B_W // LANES
SLAB_W = 100096
SLAB_H = SLAB_W // 2
CHUNK_C = 256
HCHK = 12
RING = 3
GROUPS_PER_SC = 4


def _mf_kernel(u_hbm, i_hbm, a_hbm, ub_hbm, uvt_hbm, ib_hbm, ivt_hbm,
               p_hbm, out_hbm,
               uidx_v, iidx_v, a_v, ubias_v, ibias_v,
               idxu_v, idxi_v, gu_v, gi_v, gtu_v, gti_v, part_v, p_v,
               tmpu_v, tmpi_v,
               slab_ua, slab_ub, slab_ia, slab_ib,
               sem, semst, semr, semg):
    c = lax.axis_index("c")
    s = lax.axis_index("s")
    base = s * B_W

    pltpu.sync_copy(u_hbm.at[pl.ds(base, B_W)], uidx_v)
    pltpu.sync_copy(i_hbm.at[pl.ds(base, B_W)], iidx_v)
    pltpu.sync_copy(a_hbm.at[pl.ds(base, B_W)], a_v)
    pltpu.sync_copy(p_hbm, p_v)

    cb0 = pltpu.async_copy(ub_hbm.at[uidx_v], ubias_v, sem)
    cb1 = pltpu.async_copy(ib_hbm.at[iidx_v], ibias_v, sem)
    cb0.wait()
    cb1.wait()

    pv = p_v[...]
    w0 = pv[1]
    w1 = pv[2]
    cbias = pv[0] + pv[3]
    mask = jnp.where(c == 0, 1.0, 0.0).astype(jnp.float32)

    def init_blk(blk, carry):
        sl = pl.ds(blk * LANES, LANES)
        a16 = a_v[sl]
        terms = ubias_v[sl] + ibias_v[sl] + a16 * w0 + a16 * a16 * w1 + cbias
        part_v[sl] = mask * terms
        return carry

    lax.fori_loop(0, NBLK, init_blk, 0)

    def spread_chunk(tmp, slab, col0, width):
        return [pltpu.async_copy(
            tmp.at[dd, pl.ds(0, width)],
            slab.at[pl.ds(dd * SLAB_H + col0, width)], semr)
            for dd in range(8)]

    def fire_fetch(row0, lo, t):
        hcol = pl.multiple_of(lo + CHUNK_C * s + t * (16 * CHUNK_C), 128)
        b = t % RING
        cu = pltpu.async_copy(
            uvt_hbm.at[pl.ds(row0, 8), pl.ds(hcol, CHUNK_C)],
            tmpu_v.at[b], semst)
        ci = pltpu.async_copy(
            ivt_hbm.at[pl.ds(row0, 8), pl.ds(hcol, CHUNK_C)],
            tmpi_v.at[b], semst)
        return [cu, ci]

    def stage_half(row0, lo, slab_u, slab_i):
        fetches = {t: fire_fetch(row0, lo, t) for t in range(min(RING, HCHK))}
        spreads = {}
        for t in range(HCHK):
            col0 = pl.multiple_of(CHUNK_C * s + t * (16 * CHUNK_C), 128)
            b = t % RING
            for cp in fetches.pop(t):
                cp.wait()
            spreads[t] = (spread_chunk(tmpu_v.at[b], slab_u, col0, CHUNK_C)
                          + spread_chunk(tmpi_v.at[b], slab_i, col0, CHUNK_C))
            nt = t + RING
            if nt < HCHK:
                if nt - RING in spreads:
                    for rc in spreads.pop(nt - RING):
                        rc.wait()
                fetches[nt] = fire_fetch(row0, lo, nt)
        for rcs in spreads.values():
            for rc in rcs:
                rc.wait()

        def tail_chunk(col0, width):
            cu = pltpu.async_copy(
                uvt_hbm.at[pl.ds(row0, 8),
                           pl.ds(pl.multiple_of(lo + col0, 128), width)],
                tmpu_v.at[0, :, pl.ds(0, width)], semst)
            ci = pltpu.async_copy(
                ivt_hbm.at[pl.ds(row0, 8),
                           pl.ds(pl.multiple_of(lo + col0, 128), width)],
                tmpi_v.at[0, :, pl.ds(0, width)], semst)
            cu.wait()
            rcu = spread_chunk(tmpu_v.at[0], slab_u, col0, width)
            ci.wait()
            rci = spread_chunk(tmpi_v.at[0], slab_i, col0, width)
            for rc in rcu + rci:
                rc.wait()

        @pl.when(s < 3)
        def _():
            tail_chunk(16 * HCHK * CHUNK_C + s * CHUNK_C, CHUNK_C)

        @pl.when(s == 3)
        def _():
            tail_chunk(16 * HCHK * CHUNK_C + 3 * CHUNK_C, 128)

    def build_idx(lo):
        def idx_blk(blk, carry):
            sl = pl.ds(blk * LANES, LANES)
            cu16 = jnp.clip(uidx_v[sl] - lo, 0, SLAB_H - 1)
            ci16 = jnp.clip(iidx_v[sl] - lo, 0, SLAB_H - 1)
            p0 = blk * LANES
            h = p0 >> 7
            psl = pl.ds(p0 & 127, LANES)
            for dd in range(8):
                idxu_v[dd * 2 + h, psl] = cu16 + dd * SLAB_H
                idxi_v[dd * 2 + h, psl] = ci16 + dd * SLAB_H
            return carry

        lax.fori_loop(0, NBLK, idx_blk, 0)

    def fire_gathers(slab_u, slab_i):
        return ([pltpu.async_copy(slab_u.at[idxu_v.at[k]], gtu_v.at[k], semg)
                 for k in range(16)]
                + [pltpu.async_copy(slab_i.at[idxi_v.at[k]], gti_v.at[k], semg)
                   for k in range(16)])

    def merge_h0(lo, hi):
        def blk(b2, carry):
            sl = pl.ds(b2 * LANES, LANES)
            u16 = uidx_v[sl]
            i16 = iidx_v[sl]
            mu = jnp.logical_and(u16 >= lo, u16 < hi)
            mi = jnp.logical_and(i16 >= lo, i16 < hi)
            zero = jnp.zeros((LANES,), jnp.float32)
            p0 = b2 * LANES
            h = p0 >> 7
            psl = pl.ds(p0 & 127, LANES)
            for dd in range(8):
                row = dd * 2 + h
                gu_v[row, psl] = jnp.where(mu, gtu_v[row, psl], zero)
                gi_v[row, psl] = jnp.where(mi, gti_v[row, psl], zero)
            return carry

        lax.fori_loop(0, NBLK, blk, 0)

    def merge_h1_dot(lo, hi):
        def blk(b2, carry):
            sl = pl.ds(b2 * LANES, LANES)
            u16 = uidx_v[sl]
            i16 = iidx_v[sl]
            mu = jnp.logical_and(u16 >= lo, u16 < hi)
            mi = jnp.logical_and(i16 >= lo, i16 < hi)
            acc = jnp.zeros((LANES,), jnp.float32)
            p0 = b2 * LANES
            h = p0 >> 7
            psl = pl.ds(p0 & 127, LANES)
            for dd in range(8):
                row = dd * 2 + h
                vu = jnp.where(mu, gtu_v[row, psl], gu_v[row, psl])
                vi = jnp.where(mi, gti_v[row, psl], gi_v[row, psl])
                acc = acc + vu * vi
            part_v[sl] = part_v[sl] + acc
            return carry

        lax.fori_loop(0, NBLK, blk, 0)

    def row0_of(g):
        return pl.multiple_of(8 * (GROUPS_PER_SC * c + g), 8)

    stage_half(row0_of(0), 0, slab_ua, slab_ia)
    plsc.subcore_barrier()

    def group_body(g, carry):
        build_idx(0)
        g0 = fire_gathers(slab_ua, slab_ia)
        stage_half(row0_of(g), SLAB_H, slab_ub, slab_ib)
        for cp in g0:
            cp.wait()
        merge_h0(0, SLAB_H)
        plsc.subcore_barrier()

        build_idx(SLAB_H)
        g1 = fire_gathers(slab_ub, slab_ib)

        @pl.when(g < GROUPS_PER_SC - 1)
        def _():
            stage_half(row0_of(g + 1), 0, slab_ua, slab_ia)

        for cp in g1:
            cp.wait()
        merge_h1_dot(SLAB_H, SLAB_W)
        plsc.subcore_barrier()
        return carry

    lax.fori_loop(0, GROUPS_PER_SC, group_body, 0)

    pltpu.sync_copy(part_v, out_hbm.at[pl.ds(c * BATCH + base, B_W)])


@jax.jit
def _mf(u, i, a, user_bias, user_vec, item_bias, item_vec, glob_bias, age_W, age_b):
    mesh = plsc.VectorSubcoreMesh(core_axis_name="c", subcore_axis_name="s")
    run = functools.partial(
        pl.kernel,
        mesh=mesh,
        out_type=jax.ShapeDtypeStruct((NC * BATCH,), jnp.float32),
        scratch_types=[
            pltpu.VMEM((B_W,), jnp.int32),
            pltpu.VMEM((B_W,), jnp.int32),
            pltpu.VMEM((B_W,), jnp.float32),
            pltpu.VMEM((B_W,), jnp.float32),
            pltpu.VMEM((B_W,), jnp.float32),
            pltpu.VMEM((16, 128), jnp.int32),
            pltpu.VMEM((16, 128), jnp.int32),
            pltpu.VMEM((16, 128), jnp.float32),
            pltpu.VMEM((16, 128), jnp.float32),
            pltpu.VMEM((16, 128), jnp.float32),
            pltpu.VMEM((16, 128), jnp.float32),
            pltpu.VMEM((B_W,), jnp.float32),
            pltpu.VMEM((LANES,), jnp.float32),
            pltpu.VMEM((RING, 8, CHUNK_C), jnp.float32),
            pltpu.VMEM((RING, 8, CHUNK_C), jnp.float32),
            pltpu.VMEM_SHARED((8 * SLAB_H,), jnp.float32),
            pltpu.VMEM_SHARED((8 * SLAB_H,), jnp.float32),
            pltpu.VMEM_SHARED((8 * SLAB_H,), jnp.float32),
            pltpu.VMEM_SHARED((8 * SLAB_H,), jnp.float32),
            pltpu.SemaphoreType.DMA,
            pltpu.SemaphoreType.DMA,
            pltpu.SemaphoreType.DMA,
            pltpu.SemaphoreType.DMA,
        ],
        compiler_params=pltpu.CompilerParams(
            needs_layout_passes=False, use_tc_tiling_on_sc=True),
    )(_mf_kernel)
    params = jnp.zeros((LANES,), jnp.float32)
    params = params.at[0].set(glob_bias[0, 0])
    params = params.at[1].set(age_W[0, 0])
    params = params.at[2].set(age_W[0, 1])
    params = params.at[3].set(age_b[0])
    parts = run(u, i, a, user_bias, user_vec.T, item_bias, item_vec.T, params)
    return parts[:BATCH] + parts[BATCH:]


def kernel(u, i, a, user_bias, user_vec, item_bias, item_vec, glob_bias, age_W, age_b):
    return _mf(u, i, a, user_bias, user_vec, item_bias, item_vec,
               glob_bias, age_W, age_b)


# speedup vs baseline: 1.0401x; 1.0401x over previous
import functools

import jax
import jax.numpy as jnp
from jax import lax
from jax.experimental import pallas as pl
from jax.experimental.pallas import tpu as pltpu
from jax.experimental.pallas import tpu_sc as plsc

N_ROWS = 100000
N_DIM = 64
BATCH = 4096
NC = 2
NS = 16
LANES = 16
B_W = BATCH // NS
NBLK = B_W // LANES
SLAB_W = 100096
SLAB_H = SLAB_W // 2
CHUNK_C = 256
HCHK = 12
RING = 3
GROUPS_PER_SC = 4


def _mf_kernel(u_hbm, i_hbm, a_hbm, ub_hbm, uvt_hbm, ib_hbm, ivt_hbm,
               p_hbm, out_hbm,
               uidx_v, iidx_v, a_v, ubias_v, ibias_v,
               idxu_v, idxi_v, gu_v, gi_v, gtu_v, gti_v, part_v, p_v,
               tmpu_v, tmpi_v,
               slab_ua, slab_ub, slab_ia, slab_ib,
               sem, semst, semr, semg):
    c = lax.axis_index("c")
    s = lax.axis_index("s")
    base = s * B_W

    pltpu.sync_copy(u_hbm.at[pl.ds(base, B_W)], uidx_v)
    pltpu.sync_copy(i_hbm.at[pl.ds(base, B_W)], iidx_v)
    pltpu.sync_copy(a_hbm.at[pl.ds(base, B_W)], a_v)
    pltpu.sync_copy(p_hbm, p_v)

    cb0 = pltpu.async_copy(ub_hbm.at[uidx_v], ubias_v, sem)
    cb1 = pltpu.async_copy(ib_hbm.at[iidx_v], ibias_v, sem)
    cb0.wait()
    cb1.wait()

    pv = p_v[...]
    w0 = pv[1]
    w1 = pv[2]
    cbias = pv[0] + pv[3]
    mask = jnp.where(c == 0, 1.0, 0.0).astype(jnp.float32)

    def init_blk(blk, carry):
        sl = pl.ds(blk * LANES, LANES)
        a16 = a_v[sl]
        terms = ubias_v[sl] + ibias_v[sl] + a16 * w0 + a16 * a16 * w1 + cbias
        part_v[sl] = mask * terms
        return carry

    lax.fori_loop(0, NBLK, init_blk, 0)

    def spread_chunk(tmp, slab, col0, width):
        return [pltpu.async_copy(
            tmp.at[dd, pl.ds(0, width)],
            slab.at[pl.ds(dd * SLAB_H + col0, width)], semr)
            for dd in range(8)]

    def fire_fetch(row0, lo, t):
        hcol = pl.multiple_of(lo + CHUNK_C * s, 128) + t * (16 * CHUNK_C)
        b = lax.rem(t, RING)
        pltpu.async_copy(
            uvt_hbm.at[pl.ds(row0, 8), pl.ds(hcol, CHUNK_C)],
            tmpu_v.at[b], semst)
        pltpu.async_copy(
            ivt_hbm.at[pl.ds(row0, 8), pl.ds(hcol, CHUNK_C)],
            tmpi_v.at[b], semst)

    def wait_fetch_pair():
        pltpu.make_async_copy(
            uvt_hbm.at[pl.ds(0, 8), pl.ds(0, CHUNK_C)], tmpu_v.at[0],
            semst).wait()
        pltpu.make_async_copy(
            ivt_hbm.at[pl.ds(0, 8), pl.ds(0, CHUNK_C)], tmpi_v.at[0],
            semst).wait()

    def wait_spreads(slab_u, slab_i):
        for _ in range(8):
            pltpu.make_async_copy(
                a_hbm.at[pl.ds(0, CHUNK_C)],
                slab_u.at[pl.ds(0, CHUNK_C)], semr).wait()
            pltpu.make_async_copy(
                a_hbm.at[pl.ds(0, CHUNK_C)],
                slab_i.at[pl.ds(0, CHUNK_C)], semr).wait()

    def stage_half(row0, lo, slab_u, slab_i):
        for t0 in range(RING):
            fire_fetch(row0, lo, t0)

        def round_body(t, carry):
            col0 = CHUNK_C * s + t * (16 * CHUNK_C)
            b = lax.rem(t, RING)
            wait_fetch_pair()
            spread_chunk(tmpu_v.at[b], slab_u, col0, CHUNK_C)
            spread_chunk(tmpi_v.at[b], slab_i, col0, CHUNK_C)

            @pl.when(t >= 1)
            def _():
                wait_spreads(slab_u, slab_i)

            @pl.when(jnp.logical_and(t >= 1, t + RING - 1 < HCHK))
            def _():
                fire_fetch(row0, lo, t - 1 + RING)

            return carry

        lax.fori_loop(0, HCHK, round_body, 0)
        wait_spreads(slab_u, slab_i)

        def tail_chunk(col0, width):
            cu = pltpu.async_copy(
                uvt_hbm.at[pl.ds(row0, 8),
                           pl.ds(pl.multiple_of(lo + col0, 128), width)],
                tmpu_v.at[0, :, pl.ds(0, width)], semst)
            ci = pltpu.async_copy(
                ivt_hbm.at[pl.ds(row0, 8),
                           pl.ds(pl.multiple_of(lo + col0, 128), width)],
                tmpi_v.at[0, :, pl.ds(0, width)], semst)
            cu.wait()
            rcu = spread_chunk(tmpu_v.at[0], slab_u, col0, width)
            ci.wait()
            rci = spread_chunk(tmpi_v.at[0], slab_i, col0, width)
            for rc in rcu + rci:
                rc.wait()

        @pl.when(s < 3)
        def _():
            tail_chunk(16 * HCHK * CHUNK_C + s * CHUNK_C, CHUNK_C)

        @pl.when(s == 3)
        def _():
            tail_chunk(16 * HCHK * CHUNK_C + 3 * CHUNK_C, 128)

    def build_idx(lo):
        def idx_blk(blk, carry):
            sl = pl.ds(blk * LANES, LANES)
            cu16 = jnp.clip(uidx_v[sl] - lo, 0, SLAB_H - 1)
            ci16 = jnp.clip(iidx_v[sl] - lo, 0, SLAB_H - 1)
            p0 = blk * LANES
            h = p0 >> 7
            psl = pl.ds(p0 & 127, LANES)
            for dd in range(8):
                idxu_v[dd * 2 + h, psl] = cu16 + dd * SLAB_H
                idxi_v[dd * 2 + h, psl] = ci16 + dd * SLAB_H
            return carry

        lax.fori_loop(0, NBLK, idx_blk, 0)

    def fire_gathers(slab_u, slab_i):
        return ([pltpu.async_copy(slab_u.at[idxu_v.at[k]], gtu_v.at[k], semg)
                 for k in range(16)]
                + [pltpu.async_copy(slab_i.at[idxi_v.at[k]], gti_v.at[k], semg)
                   for k in range(16)])

    def merge_h0(lo, hi):
        def blk(b2, carry):
            sl = pl.ds(b2 * LANES, LANES)
            u16 = uidx_v[sl]
            i16 = iidx_v[sl]
            mu = jnp.logical_and(u16 >= lo, u16 < hi)
            mi = jnp.logical_and(i16 >= lo, i16 < hi)
            zero = jnp.zeros((LANES,), jnp.float32)
            p0 = b2 * LANES
            h = p0 >> 7
            psl = pl.ds(p0 & 127, LANES)
            for dd in range(8):
                row = dd * 2 + h
                gu_v[row, psl] = jnp.where(mu, gtu_v[row, psl], zero)
                gi_v[row, psl] = jnp.where(mi, gti_v[row, psl], zero)
            return carry

        lax.fori_loop(0, NBLK, blk, 0)

    def merge_h1_dot(lo, hi):
        def blk(b2, carry):
            sl = pl.ds(b2 * LANES, LANES)
            u16 = uidx_v[sl]
            i16 = iidx_v[sl]
            mu = jnp.logical_and(u16 >= lo, u16 < hi)
            mi = jnp.logical_and(i16 >= lo, i16 < hi)
            acc = jnp.zeros((LANES,), jnp.float32)
            p0 = b2 * LANES
            h = p0 >> 7
            psl = pl.ds(p0 & 127, LANES)
            for dd in range(8):
                row = dd * 2 + h
                vu = jnp.where(mu, gtu_v[row, psl], gu_v[row, psl])
                vi = jnp.where(mi, gti_v[row, psl], gi_v[row, psl])
                acc = acc + vu * vi
            part_v[sl] = part_v[sl] + acc
            return carry

        lax.fori_loop(0, NBLK, blk, 0)

    def row0_of(g):
        return pl.multiple_of(8 * (GROUPS_PER_SC * c + g), 8)

    stage_half(row0_of(0), 0, slab_ua, slab_ia)
    plsc.subcore_barrier()

    def group_body(g, carry):
        build_idx(0)
        g0 = fire_gathers(slab_ua, slab_ia)
        stage_half(row0_of(g), SLAB_H, slab_ub, slab_ib)
        for cp in g0:
            cp.wait()
        merge_h0(0, SLAB_H)
        plsc.subcore_barrier()

        build_idx(SLAB_H)
        g1 = fire_gathers(slab_ub, slab_ib)

        @pl.when(g < GROUPS_PER_SC - 1)
        def _():
            stage_half(row0_of(g + 1), 0, slab_ua, slab_ia)

        for cp in g1:
            cp.wait()
        merge_h1_dot(SLAB_H, SLAB_W)
        plsc.subcore_barrier()
        return carry

    lax.fori_loop(0, GROUPS_PER_SC, group_body, 0)

    pltpu.sync_copy(part_v, out_hbm.at[pl.ds(c * BATCH + base, B_W)])


@jax.jit
def _mf(u, i, a, user_bias, user_vec, item_bias, item_vec, glob_bias, age_W, age_b):
    mesh = plsc.VectorSubcoreMesh(core_axis_name="c", subcore_axis_name="s")
    run = functools.partial(
        pl.kernel,
        mesh=mesh,
        out_type=jax.ShapeDtypeStruct((NC * BATCH,), jnp.float32),
        scratch_types=[
            pltpu.VMEM((B_W,), jnp.int32),
            pltpu.VMEM((B_W,), jnp.int32),
            pltpu.VMEM((B_W,), jnp.float32),
            pltpu.VMEM((B_W,), jnp.float32),
            pltpu.VMEM((B_W,), jnp.float32),
            pltpu.VMEM((16, 128), jnp.int32),
            pltpu.VMEM((16, 128), jnp.int32),
            pltpu.VMEM((16, 128), jnp.float32),
            pltpu.VMEM((16, 128), jnp.float32),
            pltpu.VMEM((16, 128), jnp.float32),
            pltpu.VMEM((16, 128), jnp.float32),
            pltpu.VMEM((B_W,), jnp.float32),
            pltpu.VMEM((LANES,), jnp.float32),
            pltpu.VMEM((RING, 8, CHUNK_C), jnp.float32),
            pltpu.VMEM((RING, 8, CHUNK_C), jnp.float32),
            pltpu.VMEM_SHARED((8 * SLAB_H,), jnp.float32),
            pltpu.VMEM_SHARED((8 * SLAB_H,), jnp.float32),
            pltpu.VMEM_SHARED((8 * SLAB_H,), jnp.float32),
            pltpu.VMEM_SHARED((8 * SLAB_H,), jnp.float32),
            pltpu.SemaphoreType.DMA,
            pltpu.SemaphoreType.DMA,
            pltpu.SemaphoreType.DMA,
            pltpu.SemaphoreType.DMA,
        ],
        compiler_params=pltpu.CompilerParams(
            needs_layout_passes=False, use_tc_tiling_on_sc=True),
    )(_mf_kernel)
    params = jnp.zeros((LANES,), jnp.float32)
    params = params.at[0].set(glob_bias[0, 0])
    params = params.at[1].set(age_W[0, 0])
    params = params.at[2].set(age_W[0, 1])
    params = params.at[3].set(age_b[0])
    parts = run(u, i, a, user_bias, user_vec.T, item_bias, item_vec.T, params)
    return parts[:BATCH] + parts[BATCH:]


def kernel(u, i, a, user_bias, user_vec, item_bias, item_vec, glob_bias, age_W, age_b):
    return _mf(u, i, a, user_bias, user_vec, item_bias, item_vec,
               glob_bias, age_W, age_b)


# speedup vs baseline: 1.8335x; 1.7628x over previous
import functools

import jax
import jax.numpy as jnp
from jax import lax
from jax.experimental import pallas as pl
from jax.experimental.pallas import tpu as pltpu
from jax.experimental.pallas import tpu_sc as plsc

N_ROWS = 100000
N_DIM = 64
BATCH = 4096
NC = 2
NS = 16
LANES = 16
B_W = BATCH // NS
NBLK = B_W // LANES
SLAB_W = 100096
SLAB_H = SLAB_W // 2
CHUNK_C = 256
HCHK = 12
RING = 3
GROUPS_PER_SC = 4


def _mf_kernel(u_hbm, i_hbm, a_hbm, ub_hbm, uvt_hbm, ib_hbm, ivt_hbm,
               p_hbm, out_hbm,
               uidx_v, iidx_v, a_v, ubias_v, ibias_v,
               idxu_v, idxi_v, gu_v, gi_v, gtu_v, gti_v, part_v, p_v,
               tmpu_v, tmpi_v,
               slab_ua, slab_ub, slab_ia, slab_ib,
               sem, semst, semr, semg):
    c = lax.axis_index("c")
    s = lax.axis_index("s")
    base = s * B_W

    pltpu.sync_copy(u_hbm.at[pl.ds(base, B_W)], uidx_v)
    pltpu.sync_copy(i_hbm.at[pl.ds(base, B_W)], iidx_v)
    pltpu.sync_copy(a_hbm.at[pl.ds(base, B_W)], a_v)
    pltpu.sync_copy(p_hbm, p_v)

    cb0 = pltpu.async_copy(ub_hbm.at[uidx_v], ubias_v, sem)
    cb1 = pltpu.async_copy(ib_hbm.at[iidx_v], ibias_v, sem)
    cb0.wait()
    cb1.wait()

    pv = p_v[...]
    w0 = pv[1]
    w1 = pv[2]
    cbias = pv[0] + pv[3]
    mask = jnp.where(c == 0, 1.0, 0.0).astype(jnp.float32)

    def init_blk(blk, carry):
        sl = pl.ds(blk * LANES, LANES)
        a16 = a_v[sl]
        terms = ubias_v[sl] + ibias_v[sl] + a16 * w0 + a16 * a16 * w1 + cbias
        part_v[sl] = mask * terms
        return carry

    lax.fori_loop(0, NBLK, init_blk, 0)

    def spread_chunk(tmp, slab, col0, width):
        return [pltpu.async_copy(
            tmp.at[dd, pl.ds(0, width)],
            slab.at[pl.ds(dd * SLAB_H + col0, width)], semr)
            for dd in range(8)]

    def fire_fetch(row0, lo, t):
        hcol = pl.multiple_of(lo + CHUNK_C * s, 128) + t * (16 * CHUNK_C)
        b = lax.rem(t, RING)
        pltpu.async_copy(
            uvt_hbm.at[pl.ds(row0, 8), pl.ds(hcol, CHUNK_C)],
            tmpu_v.at[b], semst)
        pltpu.async_copy(
            ivt_hbm.at[pl.ds(row0, 8), pl.ds(hcol, CHUNK_C)],
            tmpi_v.at[b], semst)

    def wait_fetch_pair():
        pltpu.make_async_copy(
            uvt_hbm.at[pl.ds(0, 8), pl.ds(0, CHUNK_C)], tmpu_v.at[0],
            semst).wait()
        pltpu.make_async_copy(
            ivt_hbm.at[pl.ds(0, 8), pl.ds(0, CHUNK_C)], tmpi_v.at[0],
            semst).wait()

    def wait_spreads(slab_u, slab_i):
        for _ in range(8):
            pltpu.make_async_copy(
                a_hbm.at[pl.ds(0, CHUNK_C)],
                slab_u.at[pl.ds(0, CHUNK_C)], semr).wait()
            pltpu.make_async_copy(
                a_hbm.at[pl.ds(0, CHUNK_C)],
                slab_i.at[pl.ds(0, CHUNK_C)], semr).wait()

    def stage_half(row0, lo, slab_u, slab_i):
        for t0 in range(RING):
            fire_fetch(row0, lo, t0)

        def round_body(t, carry):
            col0 = CHUNK_C * s + t * (16 * CHUNK_C)
            b = lax.rem(t, RING)
            wait_fetch_pair()
            spread_chunk(tmpu_v.at[b], slab_u, col0, CHUNK_C)
            spread_chunk(tmpi_v.at[b], slab_i, col0, CHUNK_C)

            @pl.when(t >= 1)
            def _():
                wait_spreads(slab_u, slab_i)

            @pl.when(jnp.logical_and(t >= 1, t + RING - 1 < HCHK))
            def _():
                fire_fetch(row0, lo, t - 1 + RING)

            return carry

        lax.fori_loop(0, HCHK, round_body, 0)
        wait_spreads(slab_u, slab_i)

        def tail_chunk(col0, width):
            cu = pltpu.async_copy(
                uvt_hbm.at[pl.ds(row0, 8),
                           pl.ds(pl.multiple_of(lo + col0, 128), width)],
                tmpu_v.at[0, :, pl.ds(0, width)], semst)
            ci = pltpu.async_copy(
                ivt_hbm.at[pl.ds(row0, 8),
                           pl.ds(pl.multiple_of(lo + col0, 128), width)],
                tmpi_v.at[0, :, pl.ds(0, width)], semst)
            cu.wait()
            rcu = spread_chunk(tmpu_v.at[0], slab_u, col0, width)
            ci.wait()
            rci = spread_chunk(tmpi_v.at[0], slab_i, col0, width)
            for rc in rcu + rci:
                rc.wait()

        @pl.when(s < 3)
        def _():
            tail_chunk(16 * HCHK * CHUNK_C + s * CHUNK_C, CHUNK_C)

        @pl.when(s == 3)
        def _():
            tail_chunk(16 * HCHK * CHUNK_C + 3 * CHUNK_C, 128)

    def build_idx(lo):
        def wrap(off):
            off = jnp.where(off < 0, off + SLAB_H, off)
            return jnp.where(off >= SLAB_H, off - SLAB_H, off)

        def idx_blk(blk, carry):
            sl = pl.ds(blk * LANES, LANES)
            cu16 = wrap(uidx_v[sl] - lo)
            ci16 = wrap(iidx_v[sl] - lo)
            p0 = blk * LANES
            h = p0 >> 7
            psl = pl.ds(p0 & 127, LANES)
            for dd in range(8):
                idxu_v[dd * 2 + h, psl] = cu16 + dd * SLAB_H
                idxi_v[dd * 2 + h, psl] = ci16 + dd * SLAB_H
            return carry

        lax.fori_loop(0, NBLK, idx_blk, 0)

    def fire_gathers(slab_u, slab_i):
        return ([pltpu.async_copy(slab_u.at[idxu_v.at[k]], gtu_v.at[k], semg)
                 for k in range(16)]
                + [pltpu.async_copy(slab_i.at[idxi_v.at[k]], gti_v.at[k], semg)
                   for k in range(16)])

    def merge_h0(lo, hi):
        def blk(b2, carry):
            sl = pl.ds(b2 * LANES, LANES)
            u16 = uidx_v[sl]
            i16 = iidx_v[sl]
            mu = jnp.logical_and(u16 >= lo, u16 < hi)
            mi = jnp.logical_and(i16 >= lo, i16 < hi)
            zero = jnp.zeros((LANES,), jnp.float32)
            p0 = b2 * LANES
            h = p0 >> 7
            psl = pl.ds(p0 & 127, LANES)
            for dd in range(8):
                row = dd * 2 + h
                gu_v[row, psl] = jnp.where(mu, gtu_v[row, psl], zero)
                gi_v[row, psl] = jnp.where(mi, gti_v[row, psl], zero)
            return carry

        lax.fori_loop(0, NBLK, blk, 0)

    def merge_h1_dot(lo, hi):
        def blk(b2, carry):
            sl = pl.ds(b2 * LANES, LANES)
            u16 = uidx_v[sl]
            i16 = iidx_v[sl]
            mu = jnp.logical_and(u16 >= lo, u16 < hi)
            mi = jnp.logical_and(i16 >= lo, i16 < hi)
            acc = jnp.zeros((LANES,), jnp.float32)
            p0 = b2 * LANES
            h = p0 >> 7
            psl = pl.ds(p0 & 127, LANES)
            for dd in range(8):
                row = dd * 2 + h
                vu = jnp.where(mu, gtu_v[row, psl], gu_v[row, psl])
                vi = jnp.where(mi, gti_v[row, psl], gi_v[row, psl])
                acc = acc + vu * vi
            part_v[sl] = part_v[sl] + acc
            return carry

        lax.fori_loop(0, NBLK, blk, 0)

    def row0_of(g):
        return pl.multiple_of(8 * (GROUPS_PER_SC * c + g), 8)

    stage_half(row0_of(0), 0, slab_ua, slab_ia)
    plsc.subcore_barrier()

    def group_body(g, carry):
        build_idx(0)
        g0 = fire_gathers(slab_ua, slab_ia)
        stage_half(row0_of(g), SLAB_H, slab_ub, slab_ib)
        for cp in g0:
            cp.wait()
        merge_h0(0, SLAB_H)
        plsc.subcore_barrier()

        build_idx(SLAB_H)
        g1 = fire_gathers(slab_ub, slab_ib)

        @pl.when(g < GROUPS_PER_SC - 1)
        def _():
            stage_half(row0_of(g + 1), 0, slab_ua, slab_ia)

        for cp in g1:
            cp.wait()
        merge_h1_dot(SLAB_H, SLAB_W)
        plsc.subcore_barrier()
        return carry

    lax.fori_loop(0, GROUPS_PER_SC, group_body, 0)

    pltpu.sync_copy(part_v, out_hbm.at[pl.ds(c * BATCH + base, B_W)])


@jax.jit
def _mf(u, i, a, user_bias, user_vec, item_bias, item_vec, glob_bias, age_W, age_b):
    mesh = plsc.VectorSubcoreMesh(core_axis_name="c", subcore_axis_name="s")
    run = functools.partial(
        pl.kernel,
        mesh=mesh,
        out_type=jax.ShapeDtypeStruct((NC * BATCH,), jnp.float32),
        scratch_types=[
            pltpu.VMEM((B_W,), jnp.int32),
            pltpu.VMEM((B_W,), jnp.int32),
            pltpu.VMEM((B_W,), jnp.float32),
            pltpu.VMEM((B_W,), jnp.float32),
            pltpu.VMEM((B_W,), jnp.float32),
            pltpu.VMEM((16, 128), jnp.int32),
            pltpu.VMEM((16, 128), jnp.int32),
            pltpu.VMEM((16, 128), jnp.float32),
            pltpu.VMEM((16, 128), jnp.float32),
            pltpu.VMEM((16, 128), jnp.float32),
            pltpu.VMEM((16, 128), jnp.float32),
            pltpu.VMEM((B_W,), jnp.float32),
            pltpu.VMEM((LANES,), jnp.float32),
            pltpu.VMEM((RING, 8, CHUNK_C), jnp.float32),
            pltpu.VMEM((RING, 8, CHUNK_C), jnp.float32),
            pltpu.VMEM_SHARED((8 * SLAB_H,), jnp.float32),
            pltpu.VMEM_SHARED((8 * SLAB_H,), jnp.float32),
            pltpu.VMEM_SHARED((8 * SLAB_H,), jnp.float32),
            pltpu.VMEM_SHARED((8 * SLAB_H,), jnp.float32),
            pltpu.SemaphoreType.DMA,
            pltpu.SemaphoreType.DMA,
            pltpu.SemaphoreType.DMA,
            pltpu.SemaphoreType.DMA,
        ],
        compiler_params=pltpu.CompilerParams(
            needs_layout_passes=False, use_tc_tiling_on_sc=True),
    )(_mf_kernel)
    params = jnp.zeros((LANES,), jnp.float32)
    params = params.at[0].set(glob_bias[0, 0])
    params = params.at[1].set(age_W[0, 0])
    params = params.at[2].set(age_W[0, 1])
    params = params.at[3].set(age_b[0])
    parts = run(u, i, a, user_bias, user_vec.T, item_bias, item_vec.T, params)
    return parts[:BATCH] + parts[BATCH:]


def kernel(u, i, a, user_bias, user_vec, item_bias, item_vec, glob_bias, age_W, age_b):
    return _mf(u, i, a, user_bias, user_vec, item_bias, item_vec,
               glob_bias, age_W, age_b)


# speedup vs baseline: 2.0972x; 1.1439x over previous
import functools

import jax
import jax.numpy as jnp
from jax import lax
from jax.experimental import pallas as pl
from jax.experimental.pallas import tpu as pltpu
from jax.experimental.pallas import tpu_sc as plsc

N_ROWS = 100000
N_DIM = 64
BATCH = 4096
NC = 2
NS = 16
LANES = 16
B_W = BATCH // NS
NBLK = B_W // LANES
SLAB_W = 100096
CHUNK_C = 256
NCHK = 24
RING = 3
GROUPS_PER_SC = 4


def _mf_kernel(u_hbm, i_hbm, a_hbm, ub_hbm, uvt_hbm, ib_hbm, ivt_hbm,
               p_hbm, out_hbm,
               uidx_v, iidx_v, a_v, ubias_v, ibias_v,
               idxu_v, idxi_v, gu_v, gi_v, part_v, p_v,
               tmpu_v, tmpi_v, slab_u, slab_i,
               sem, semst, semr, semg):
    c = lax.axis_index("c")
    s = lax.axis_index("s")
    base = s * B_W

    pltpu.sync_copy(u_hbm.at[pl.ds(base, B_W)], uidx_v)
    pltpu.sync_copy(i_hbm.at[pl.ds(base, B_W)], iidx_v)
    pltpu.sync_copy(a_hbm.at[pl.ds(base, B_W)], a_v)
    pltpu.sync_copy(p_hbm, p_v)

    cb0 = pltpu.async_copy(ub_hbm.at[uidx_v], ubias_v, sem)
    cb1 = pltpu.async_copy(ib_hbm.at[iidx_v], ibias_v, sem)
    cb0.wait()
    cb1.wait()

    pv = p_v[...]
    w0 = pv[1]
    w1 = pv[2]
    cbias = pv[0] + pv[3]
    mask = jnp.where(c == 0, 1.0, 0.0).astype(jnp.float32)

    def init_blk(blk, carry):
        sl = pl.ds(blk * LANES, LANES)
        a16 = a_v[sl]
        terms = ubias_v[sl] + ibias_v[sl] + a16 * w0 + a16 * a16 * w1 + cbias
        part_v[sl] = mask * terms
        return carry

    lax.fori_loop(0, NBLK, init_blk, 0)

    def idx_blk(blk, carry):
        sl = pl.ds(blk * LANES, LANES)
        u16 = uidx_v[sl]
        i16 = iidx_v[sl]
        p0 = blk * LANES
        h = p0 >> 7
        psl = pl.ds(p0 & 127, LANES)
        for dd in range(8):
            idxu_v[dd * 2 + h, psl] = u16 + dd * SLAB_W
            idxi_v[dd * 2 + h, psl] = i16 + dd * SLAB_W
        return carry

    lax.fori_loop(0, NBLK, idx_blk, 0)

    def spread_chunk(tmp, slab, col0, width):
        for dd in range(8):
            pltpu.async_copy(
                tmp.at[dd, pl.ds(0, width)],
                slab.at[pl.ds(dd * SLAB_W + col0, width)], semr)

    def fire_fetch(row0, t):
        hcol = pl.multiple_of(CHUNK_C * s, 128) + t * (16 * CHUNK_C)
        b = lax.rem(t, RING)
        pltpu.async_copy(
            uvt_hbm.at[pl.ds(row0, 8), pl.ds(hcol, CHUNK_C)],
            tmpu_v.at[b], semst)
        pltpu.async_copy(
            ivt_hbm.at[pl.ds(row0, 8), pl.ds(hcol, CHUNK_C)],
            tmpi_v.at[b], semst)

    def wait_fetch_pair():
        pltpu.make_async_copy(
            uvt_hbm.at[pl.ds(0, 8), pl.ds(0, CHUNK_C)], tmpu_v.at[0],
            semst).wait()
        pltpu.make_async_copy(
            ivt_hbm.at[pl.ds(0, 8), pl.ds(0, CHUNK_C)], tmpi_v.at[0],
            semst).wait()

    def wait_spreads():
        for _ in range(8):
            pltpu.make_async_copy(
                a_hbm.at[pl.ds(0, CHUNK_C)],
                slab_u.at[pl.ds(0, CHUNK_C)], semr).wait()
            pltpu.make_async_copy(
                a_hbm.at[pl.ds(0, CHUNK_C)],
                slab_i.at[pl.ds(0, CHUNK_C)], semr).wait()

    def stage_slabs(row0):
        for t0 in range(RING):
            fire_fetch(row0, t0)

        def round_body(t, carry):
            col0 = CHUNK_C * s + t * (16 * CHUNK_C)
            b = lax.rem(t, RING)
            wait_fetch_pair()
            spread_chunk(tmpu_v.at[b], slab_u, col0, CHUNK_C)
            spread_chunk(tmpi_v.at[b], slab_i, col0, CHUNK_C)

            @pl.when(t >= 1)
            def _():
                wait_spreads()

            @pl.when(jnp.logical_and(t >= 1, t + RING - 1 < NCHK))
            def _():
                fire_fetch(row0, t - 1 + RING)

            return carry

        lax.fori_loop(0, NCHK, round_body, 0)
        wait_spreads()

        @pl.when(s < 7)
        def _():
            col0 = 16 * NCHK * CHUNK_C + s * CHUNK_C
            cu = pltpu.async_copy(
                uvt_hbm.at[pl.ds(row0, 8),
                           pl.ds(pl.multiple_of(col0, 128), CHUNK_C)],
                tmpu_v.at[0], semst)
            ci = pltpu.async_copy(
                ivt_hbm.at[pl.ds(row0, 8),
                           pl.ds(pl.multiple_of(col0, 128), CHUNK_C)],
                tmpi_v.at[0], semst)
            cu.wait()
            spread_chunk(tmpu_v.at[0], slab_u, col0, CHUNK_C)
            ci.wait()
            spread_chunk(tmpi_v.at[0], slab_i, col0, CHUNK_C)
            wait_spreads()

    def group_body(g, carry):
        row0 = pl.multiple_of(8 * (GROUPS_PER_SC * c + g), 8)
        stage_slabs(row0)
        plsc.subcore_barrier()

        gcopies = []
        for k in range(16):
            gcopies.append(pltpu.async_copy(
                slab_u.at[idxu_v.at[k]], gu_v.at[k], semg))
            gcopies.append(pltpu.async_copy(
                slab_i.at[idxi_v.at[k]], gi_v.at[k], semg))
        for cp in gcopies:
            cp.wait()

        def acc_blk(blk, carry2):
            sl = pl.ds(blk * LANES, LANES)
            acc = jnp.zeros((LANES,), jnp.float32)
            p0 = blk * LANES
            h = p0 >> 7
            psl = pl.ds(p0 & 127, LANES)
            for dd in range(8):
                row = dd * 2 + h
                acc = acc + gu_v[row, psl] * gi_v[row, psl]
            part_v[sl] = part_v[sl] + acc
            return carry2

        lax.fori_loop(0, NBLK, acc_blk, 0)

        plsc.subcore_barrier()
        return carry

    lax.fori_loop(0, GROUPS_PER_SC, group_body, 0)

    pltpu.sync_copy(part_v, out_hbm.at[pl.ds(c * BATCH + base, B_W)])


@jax.jit
def _mf(u, i, a, user_bias, user_vec, item_bias, item_vec, glob_bias, age_W, age_b):
    mesh = plsc.VectorSubcoreMesh(core_axis_name="c", subcore_axis_name="s")
    run = functools.partial(
        pl.kernel,
        mesh=mesh,
        out_type=jax.ShapeDtypeStruct((NC * BATCH,), jnp.float32),
        scratch_types=[
            pltpu.VMEM((B_W,), jnp.int32),
            pltpu.VMEM((B_W,), jnp.int32),
            pltpu.VMEM((B_W,), jnp.float32),
            pltpu.VMEM((B_W,), jnp.float32),
            pltpu.VMEM((B_W,), jnp.float32),
            pltpu.VMEM((16, 128), jnp.int32),
            pltpu.VMEM((16, 128), jnp.int32),
            pltpu.VMEM((16, 128), jnp.float32),
            pltpu.VMEM((16, 128), jnp.float32),
            pltpu.VMEM((B_W,), jnp.float32),
            pltpu.VMEM((LANES,), jnp.float32),
            pltpu.VMEM((RING, 8, CHUNK_C), jnp.float32),
            pltpu.VMEM((RING, 8, CHUNK_C), jnp.float32),
            pltpu.VMEM_SHARED((8 * SLAB_W,), jnp.float32),
            pltpu.VMEM_SHARED((8 * SLAB_W,), jnp.float32),
            pltpu.SemaphoreType.DMA,
            pltpu.SemaphoreType.DMA,
            pltpu.SemaphoreType.DMA,
            pltpu.SemaphoreType.DMA,
        ],
        compiler_params=pltpu.CompilerParams(
            needs_layout_passes=False, use_tc_tiling_on_sc=True),
    )(_mf_kernel)
    params = jnp.zeros((LANES,), jnp.float32)
    params = params.at[0].set(glob_bias[0, 0])
    params = params.at[1].set(age_W[0, 0])
    params = params.at[2].set(age_W[0, 1])
    params = params.at[3].set(age_b[0])
    parts = run(u, i, a, user_bias, user_vec.T, item_bias, item_vec.T, params)
    return parts[:BATCH] + parts[BATCH:]


def kernel(u, i, a, user_bias, user_vec, item_bias, item_vec, glob_bias, age_W, age_b):
    return _mf(u, i, a, user_bias, user_vec, item_bias, item_vec,
               glob_bias, age_W, age_b)


# speedup vs baseline: 2.2744x; 1.0845x over previous
import functools

import jax
import jax.numpy as jnp
from jax import lax
from jax.experimental import pallas as pl
from jax.experimental.pallas import tpu as pltpu
from jax.experimental.pallas import tpu_sc as plsc

N_ROWS = 100000
N_DIM = 64
BATCH = 4096
NC = 2
NS = 16
LANES = 16
B_W = BATCH // NS
NBLK = B_W // LANES
NQP = 782
SLAB_W = NQP * 128
CHUNK_C = 256
NCHK = 24
RING = 4
GROUPS_PER_SC = 4


def _mf_kernel(u_hbm, i_hbm, a_hbm, ub_hbm, uvt_hbm, ib_hbm, ivt_hbm,
               p_hbm, out_hbm,
               uidx_v, iidx_v, a_v, ubias_v, ibias_v,
               idxu_v, idxi_v, gu_v, gi_v, part_v, p_v,
               tmpu_v, tmpi_v, slab_u, slab_i,
               sem, semst, semr, semg):
    c = lax.axis_index("c")
    s = lax.axis_index("s")
    base = s * B_W

    pltpu.sync_copy(u_hbm.at[pl.ds(base, B_W)], uidx_v)
    pltpu.sync_copy(i_hbm.at[pl.ds(base, B_W)], iidx_v)
    pltpu.sync_copy(a_hbm.at[pl.ds(base, B_W)], a_v)
    pltpu.sync_copy(p_hbm, p_v)

    cb0 = pltpu.async_copy(ub_hbm.at[uidx_v], ubias_v, sem)
    cb1 = pltpu.async_copy(ib_hbm.at[iidx_v], ibias_v, sem)
    cb0.wait()
    cb1.wait()

    def idx_blk(blk, carry):
        sl = pl.ds(blk * LANES, LANES)
        u16 = uidx_v[sl]
        i16 = iidx_v[sl]
        p0 = blk * LANES
        h = p0 >> 7
        psl = pl.ds(p0 & 127, LANES)
        for dd in range(8):
            idxu_v[dd * 2 + h, psl] = u16 + dd * SLAB_W
            idxi_v[dd * 2 + h, psl] = i16 + dd * SLAB_W
        return carry

    lax.fori_loop(0, NBLK, idx_blk, 0)

    def spread_chunk(tbl_hbm, tmp_v, slab, row0, col0, width):
        rcopies = []
        for dd in range(8):
            rcopies.append(pltpu.async_copy(
                tmp_v.at[dd, pl.ds(0, width)],
                slab.at[pl.ds(dd * SLAB_W + col0, width)], semr))
        return rcopies

    def fire_fetch(row0, t):
        col0 = pl.multiple_of(CHUNK_C * s + t * (16 * CHUNK_C), 128)
        b = t % RING
        cu = pltpu.async_copy(
            uvt_hbm.at[pl.ds(row0, 8), pl.ds(col0, CHUNK_C)],
            tmpu_v.at[b], semst)
        ci = pltpu.async_copy(
            ivt_hbm.at[pl.ds(row0, 8), pl.ds(col0, CHUNK_C)],
            tmpi_v.at[b], semst)
        return [cu, ci]

    def stage_slabs(row0):
        fetches = {t: fire_fetch(row0, t) for t in range(min(RING, NCHK))}
        spreads = {}
        for t in range(NCHK):
            col0 = pl.multiple_of(CHUNK_C * s + t * (16 * CHUNK_C), 128)
            b = t % RING
            for cp in fetches.pop(t):
                cp.wait()
            spreads[t] = (
                spread_chunk(uvt_hbm, tmpu_v.at[b], slab_u, row0, col0, CHUNK_C)
                + spread_chunk(ivt_hbm, tmpi_v.at[b], slab_i, row0, col0, CHUNK_C))
            nt = t + RING
            if nt < NCHK:
                if nt - RING in spreads:
                    for rc in spreads.pop(nt - RING):
                        rc.wait()
                fetches[nt] = fire_fetch(row0, nt)
        for rcs in spreads.values():
            for rc in rcs:
                rc.wait()

        def tail_chunk(col0, width):
            cu = pltpu.async_copy(
                uvt_hbm.at[pl.ds(row0, 8), pl.ds(col0, width)],
                tmpu_v.at[0, :, pl.ds(0, width)], semst)
            ci = pltpu.async_copy(
                ivt_hbm.at[pl.ds(row0, 8), pl.ds(col0, width)],
                tmpi_v.at[0, :, pl.ds(0, width)], semst)
            cu.wait()
            rcu = spread_chunk(uvt_hbm, tmpu_v.at[0], slab_u, row0, col0, width)
            ci.wait()
            rci = spread_chunk(ivt_hbm, tmpi_v.at[0], slab_i, row0, col0, width)
            for rc in rcu + rci:
                rc.wait()

        @pl.when(s < 7)
        def _():
            tail_chunk(pl.multiple_of(16 * NCHK * CHUNK_C + s * CHUNK_C, 128),
                       CHUNK_C)

    def gather_slab(slab, idx_v, g_v):
        gcopies = []
        for k in range(16):
            gcopies.append(pltpu.async_copy(
                slab.at[idx_v.at[k]], g_v.at[k], semg))
        for cp in gcopies:
            cp.wait()

    pv = p_v[...]
    w0 = pv[1]
    w1 = pv[2]
    cbias = pv[0] + pv[3]
    mask = jnp.where(c == 0, 1.0, 0.0).astype(jnp.float32)

    def init_blk(blk, carry):
        sl = pl.ds(blk * LANES, LANES)
        a16 = a_v[sl]
        terms = ubias_v[sl] + ibias_v[sl] + a16 * w0 + a16 * a16 * w1 + cbias
        part_v[sl] = mask * terms
        return carry

    lax.fori_loop(0, NBLK, init_blk, 0)

    def group_body(g, carry):
        row0 = pl.multiple_of(8 * (GROUPS_PER_SC * c + g), 8)

        stage_slabs(row0)
        plsc.subcore_barrier()
        gather_slab(slab_u, idxu_v, gu_v)
        gather_slab(slab_i, idxi_v, gi_v)

        def acc_blk(blk, carry2):
            sl = pl.ds(blk * LANES, LANES)
            acc = jnp.zeros((LANES,), jnp.float32)
            p0 = blk * LANES
            h = p0 >> 7
            psl = pl.ds(p0 & 127, LANES)
            for dd in range(8):
                row = dd * 2 + h
                acc = acc + gu_v[row, psl] * gi_v[row, psl]
            part_v[sl] = part_v[sl] + acc
            return carry2

        lax.fori_loop(0, NBLK, acc_blk, 0)

        plsc.subcore_barrier()
        return carry

    lax.fori_loop(0, GROUPS_PER_SC, group_body, 0)

    pltpu.sync_copy(part_v, out_hbm.at[pl.ds(c * BATCH + base, B_W)])


@jax.jit
def _mf(u, i, a, user_bias, user_vec, item_bias, item_vec, glob_bias, age_W, age_b):
    mesh = plsc.VectorSubcoreMesh(core_axis_name="c", subcore_axis_name="s")
    run = functools.partial(
        pl.kernel,
        mesh=mesh,
        out_type=jax.ShapeDtypeStruct((NC * BATCH,), jnp.float32),
        scratch_types=[
            pltpu.VMEM((B_W,), jnp.int32),
            pltpu.VMEM((B_W,), jnp.int32),
            pltpu.VMEM((B_W,), jnp.float32),
            pltpu.VMEM((B_W,), jnp.float32),
            pltpu.VMEM((B_W,), jnp.float32),
            pltpu.VMEM((16, 128), jnp.int32),
            pltpu.VMEM((16, 128), jnp.int32),
            pltpu.VMEM((16, 128), jnp.float32),
            pltpu.VMEM((16, 128), jnp.float32),
            pltpu.VMEM((B_W,), jnp.float32),
            pltpu.VMEM((LANES,), jnp.float32),
            pltpu.VMEM((RING, 8, CHUNK_C), jnp.float32),
            pltpu.VMEM((RING, 8, CHUNK_C), jnp.float32),
            pltpu.VMEM_SHARED((8 * SLAB_W,), jnp.float32),
            pltpu.VMEM_SHARED((8 * SLAB_W,), jnp.float32),
            pltpu.SemaphoreType.DMA,
            pltpu.SemaphoreType.DMA,
            pltpu.SemaphoreType.DMA,
            pltpu.SemaphoreType.DMA,
        ],
        compiler_params=pltpu.CompilerParams(
            needs_layout_passes=False, use_tc_tiling_on_sc=True),
    )(_mf_kernel)
    params = jnp.zeros((LANES,), jnp.float32)
    params = params.at[0].set(glob_bias[0, 0])
    params = params.at[1].set(age_W[0, 0])
    params = params.at[2].set(age_W[0, 1])
    params = params.at[3].set(age_b[0])
    parts = run(u, i, a, user_bias, user_vec.T, item_bias, item_vec.T, params)
    return parts[:BATCH] + parts[BATCH:]


def kernel(u, i, a, user_bias, user_vec, item_bias, item_vec, glob_bias, age_W, age_b):
    return _mf(u, i, a, user_bias, user_vec, item_bias, item_vec,
               glob_bias, age_W, age_b)
